# baseline (speedup 1.0000x reference)
import functools

import jax
from jax import lax
from jax.experimental import pallas as pl
from jax.experimental.pallas import tpu as pltpu
from jax.experimental.pallas import tpu_sc as plsc

_GRID = 10
_CHUNKS = 10


def _xu_body(x_ref, u_ref, xo_ref, uo_ref):
    xo_ref[...] = x_ref[...]
    uo_ref[...] = u_ref[...]


def _copy_xu(x, u):
    n_x = x.shape[0] // _GRID
    xs = pl.BlockSpec((n_x, x.shape[1]), lambda i: (i, 0))
    us = pl.BlockSpec(u.shape, lambda i: (0, 0))
    return pl.pallas_call(
        _xu_body,
        grid=(_GRID,),
        out_shape=(
            jax.ShapeDtypeStruct(x.shape, x.dtype),
            jax.ShapeDtypeStruct(u.shape, u.dtype),
        ),
        in_specs=[xs, us],
        out_specs=(xs, us),
    )(x, u)


def _copy_ea(edge_attr):
    info = plsc.get_sparse_core_info()
    n_workers = info.num_cores * info.num_subcores
    rows_w = edge_attr.shape[0] // n_workers
    rows_c = rows_w // _CHUNKS
    mesh = plsc.VectorSubcoreMesh(core_axis_name="c", subcore_axis_name="s")

    @functools.partial(
        pl.kernel,
        mesh=mesh,
        out_type=jax.ShapeDtypeStruct(edge_attr.shape, edge_attr.dtype),
        scratch_types=[
            pltpu.VMEM((rows_c, edge_attr.shape[1]), edge_attr.dtype),
        ],
        compiler_params=pltpu.CompilerParams(use_tc_tiling_on_sc=False),
    )
    def _ea_kernel(ea_hbm, out_hbm, buf):
        wid = lax.axis_index("s") * info.num_cores + lax.axis_index("c")
        base = wid * rows_w
        for k in range(_CHUNKS):
            start = pl.multiple_of(base + k * rows_c, 8)
            pltpu.sync_copy(ea_hbm.at[pl.ds(start, rows_c), :], buf)
            pltpu.sync_copy(buf, out_hbm.at[pl.ds(start, rows_c), :])

    return _ea_kernel(edge_attr)


def kernel(x, edge_index, edge_attr, u, batch, queries, num_props):
    xo, uo = _copy_xu(x, u)
    eao = _copy_ea(edge_attr)
    return (xo, eao, uo)


# speedup vs baseline: 1.0863x; 1.0863x over previous
import jax
from jax.experimental import pallas as pl
from jax.experimental.pallas import tpu as pltpu

_NQ = 4
_EA_CHUNK = 4000
_X_CHUNK = 5000


def _body(ea_in, x_in, u_in, ea_out, x_out, u_out,
          ea_bufs, x_bufs, ea_sin, ea_sout, x_sin, x_sout):
    u_out[...] = u_in[...]

    rows = ea_in.shape[0]
    slab = rows // _NQ
    n_c = slab // _EA_CHUNK

    def cin(q, c):
        b = 2 * q + c % 2
        return pltpu.make_async_copy(
            ea_in.at[pl.ds(q * slab + c * _EA_CHUNK, _EA_CHUNK), :],
            ea_bufs[b], ea_sin[b])

    def cout(q, c):
        b = 2 * q + c % 2
        return pltpu.make_async_copy(
            ea_bufs[b],
            ea_out.at[pl.ds(q * slab + c * _EA_CHUNK, _EA_CHUNK), :],
            ea_sout[b])

    def x_cin(c):
        return pltpu.make_async_copy(
            x_in.at[pl.ds(c * _X_CHUNK, _X_CHUNK), :], x_bufs[c], x_sin[c])

    def x_cout(c):
        return pltpu.make_async_copy(
            x_bufs[c], x_out.at[pl.ds(c * _X_CHUNK, _X_CHUNK), :], x_sout[c])

    x_cin(0).start()
    x_cin(1).start()
    for q in range(_NQ):
        cin(q, 0).start()
    x_cin(0).wait()
    x_cout(0).start()
    x_cin(1).wait()
    x_cout(1).start()
    for c in range(n_c):
        for q in range(_NQ):
            cin(q, c).wait()
            cout(q, c).start()
        for q in range(_NQ):
            if c + 1 < n_c:
                if c >= 1:
                    cout(q, c - 1).wait()
                cin(q, c + 1).start()
    for q in range(_NQ):
        if n_c >= 2:
            cout(q, n_c - 2).wait()
        cout(q, n_c - 1).wait()
    x_cout(0).wait()
    x_cout(1).wait()


def kernel(x, edge_index, edge_attr, u, batch, queries, num_props):
    any_spec = pl.BlockSpec(memory_space=pl.ANY)
    u_spec = pl.BlockSpec(u.shape, lambda: (0, 0))
    outs = pl.pallas_call(
        _body,
        out_shape=(
            jax.ShapeDtypeStruct(edge_attr.shape, edge_attr.dtype),
            jax.ShapeDtypeStruct(x.shape, x.dtype),
            jax.ShapeDtypeStruct(u.shape, u.dtype),
        ),
        in_specs=[any_spec, any_spec, u_spec],
        out_specs=(any_spec, any_spec, u_spec),
        scratch_shapes=[
            [pltpu.VMEM((_EA_CHUNK, edge_attr.shape[1]), edge_attr.dtype)
             for _ in range(2 * _NQ)],
            [pltpu.VMEM((_X_CHUNK, x.shape[1]), x.dtype) for _ in range(2)],
            [pltpu.SemaphoreType.DMA for _ in range(2 * _NQ)],
            [pltpu.SemaphoreType.DMA for _ in range(2 * _NQ)],
            [pltpu.SemaphoreType.DMA for _ in range(2)],
            [pltpu.SemaphoreType.DMA for _ in range(2)],
        ],
        compiler_params=pltpu.CompilerParams(
            vmem_limit_bytes=100 * 1024 * 1024,
        ),
    )(edge_attr, x, u)
    return (outs[1], outs[0], outs[2])


# speedup vs baseline: 16.2940x; 14.9997x over previous
import jax
from jax.experimental import pallas as pl

_GRID = 10


def _copy_body(ea_ref, x_ref, u_ref, eao_ref, xo_ref, uo_ref):
    eao_ref[...] = ea_ref[...]
    xo_ref[...] = x_ref[...]
    uo_ref[...] = u_ref[...]


def kernel(x, edge_index, edge_attr, u, batch, queries, num_props):
    ea_t = edge_attr.T
    n_ea = ea_t.shape[1] // _GRID
    n_x = x.shape[0] // _GRID
    eas = pl.BlockSpec((ea_t.shape[0], n_ea), lambda i: (0, i))
    xs = pl.BlockSpec((n_x, x.shape[1]), lambda i: (i, 0))
    us = pl.BlockSpec(u.shape, lambda i: (0, 0))
    outs = pl.pallas_call(
        _copy_body,
        grid=(_GRID,),
        out_shape=(
            jax.ShapeDtypeStruct(ea_t.shape, ea_t.dtype),
            jax.ShapeDtypeStruct(x.shape, x.dtype),
            jax.ShapeDtypeStruct(u.shape, u.dtype),
        ),
        in_specs=[eas, xs, us],
        out_specs=(eas, xs, us),
    )(ea_t, x, u)
    return (outs[1], outs[0].T, outs[2])
